# baseline (speedup 1.0000x reference)
import functools
import math

import jax
import jax.numpy as jnp
from jax import lax
from jax.experimental import pallas as pl
from jax.experimental.pallas import tpu as pltpu
from jax.experimental.pallas import tpu_sc as plsc

D = 768; H = 12; DH = 64; DC = 192; DR = 64; HID = 1024
E = 8; SH = 1; V = 8192; THETA = 10000.0; SCALE = 0.02
NM = 6; TGT = 10; S = 2048
NMOD = NM + 1
DQK = DH + DR
NP = H // 2

_F32 = jnp.float32


def _sc_gather7(tables, seq):
    nw = 32
    ch = S // nw

    @functools.partial(
        pl.kernel,
        out_type=jax.ShapeDtypeStruct((NMOD * S, D), _F32),
        mesh=plsc.VectorSubcoreMesh(core_axis_name="c", subcore_axis_name="s"),
        scratch_types=[
            pltpu.VMEM((ch,), jnp.int32),
            pltpu.VMEM((ch, D), _F32),
            pltpu.SemaphoreType.DMA,
        ],
    )
    def k(t0, t1, t2, t3, t4, t5, t6, seq_hbm, out_hbm, idx_v, rows_v, sem):
        wid = lax.axis_index("s") * 2 + lax.axis_index("c")
        base = wid * ch
        pltpu.sync_copy(seq_hbm.at[pl.ds(base, ch)], idx_v)
        for m, t in enumerate((t0, t1, t2, t3, t4, t5, t6)):
            pltpu.async_copy(t.at[idx_v], rows_v, sem).wait()
            pltpu.sync_copy(rows_v, out_hbm.at[pl.ds(m * S + base, ch)])

    return k(*tables, seq)


def _bmm(x, w, bn, x_cols=None):
    g, m = x.shape[0], x.shape[1]
    kd = w.shape[1]
    n = w.shape[2]
    xoff = 0 if x_cols is None else x_cols[0] // kd

    def body(x_ref, w_ref, o_ref):
        o_ref[0] = lax.dot_general(
            x_ref[0], w_ref[0], (((1,), (0,)), ((), ())),
            preferred_element_type=_F32)

    return pl.pallas_call(
        body,
        grid=(g, n // bn),
        in_specs=[
            pl.BlockSpec((1, m, kd), lambda i, j: (i, 0, xoff)),
            pl.BlockSpec((1, kd, bn), lambda i, j: (i, 0, j)),
        ],
        out_specs=pl.BlockSpec((1, m, bn), lambda i, j: (i, 0, j)),
        out_shape=jax.ShapeDtypeStruct((g, m, n), _F32),
        compiler_params=pltpu.CompilerParams(
            dimension_semantics=("parallel", "parallel")),
    )(x, w)


def _rot_pairs(x):
    r1 = jnp.roll(x, 1, axis=-1)
    r2 = jnp.roll(x, -1, axis=-1)
    lane = lax.broadcasted_iota(jnp.int32, x.shape, len(x.shape) - 1)
    return jnp.where(lane % 2 == 0, -r2, r1)


def _k_assemble(p1, p2, cosr, sinr):
    def body(kr_ref, kn_ref, c_ref, s_ref, o_ref):
        kr = kr_ref[0][:, :DR]
        kr_ro = kr * c_ref[...] + _rot_pairs(kr) * s_ref[...]
        kn = kn_ref[0]
        parts = []
        for h in range(H):
            parts.append(kn[:, h * DH:(h + 1) * DH])
            parts.append(kr_ro)
        o_ref[0] = jnp.concatenate(parts, axis=-1)

    return pl.pallas_call(
        body,
        grid=(NMOD,),
        in_specs=[
            pl.BlockSpec((1, S, 128), lambda i: (i, 0, 14)),
            pl.BlockSpec((1, S, H * DH), lambda i: (i, 0, 0)),
            pl.BlockSpec((S, DR), lambda i: (0, 0)),
            pl.BlockSpec((S, DR), lambda i: (0, 0)),
        ],
        out_specs=pl.BlockSpec((1, S, H * DQK), lambda i: (i, 0, 0)),
        out_shape=jax.ShapeDtypeStruct((NMOD, S, H * DQK), _F32),
        compiler_params=pltpu.CompilerParams(
            dimension_semantics=("parallel",)),
    )(p1, p2, cosr, sinr)


_BQ = 256


def _attention(p1, kf, p2, cosq, sinq):
    bq = _BQ
    scale = 1.0 / math.sqrt(float(DQK))

    def body(q_ref, k_ref, v_ref, c_ref, s_ref, o_ref):
        qb = pl.program_id(2)
        q = q_ref[0]
        q = q * c_ref[...] + _rot_pairs(q) * s_ref[...]
        row = lax.broadcasted_iota(jnp.int32, (bq, bq), 0)
        col = lax.broadcasted_iota(jnp.int32, (bq, bq), 1)
        outs = []
        for t in range(2):
            qh = q[:, t * DQK:(t + 1) * DQK]

            def step(c, carry):
                mr, lr, acc = carry
                kc = k_ref[0, pl.ds(c * bq, bq), t * DQK:(t + 1) * DQK]
                sc = lax.dot_general(
                    qh, kc, (((1,), (1,)), ((), ())),
                    preferred_element_type=_F32) * scale
                sc = jnp.where((c - qb) * bq + col <= row, sc, -1e9)
                mn = jnp.maximum(mr, jnp.max(sc, -1, keepdims=True))
                corr = jnp.exp(mr - mn)
                p = jnp.exp(sc - mn)
                vc = v_ref[0, pl.ds(c * bq, bq), t * DH:(t + 1) * DH]
                acc = acc * corr + lax.dot_general(
                    p, vc, (((1,), (0,)), ((), ())),
                    preferred_element_type=_F32)
                return mn, lr * corr + jnp.sum(p, -1, keepdims=True), acc

            mr, lr, acc = lax.fori_loop(
                0, qb + 1, step,
                (jnp.full((bq, 1), -1e30, _F32),
                 jnp.zeros((bq, 1), _F32),
                 jnp.zeros((bq, DH), _F32)))
            outs.append(acc / lr)
        o_ref[0] = jnp.concatenate(outs, axis=-1)

    return pl.pallas_call(
        body,
        grid=(NMOD, NP, S // bq),
        in_specs=[
            pl.BlockSpec((1, bq, 2 * DQK), lambda m, hp, qb: (m, qb, hp)),
            pl.BlockSpec((1, S, 2 * DQK), lambda m, hp, qb: (m, 0, hp)),
            pl.BlockSpec((1, S, 2 * DH), lambda m, hp, qb: (m, 0, 6 + hp)),
            pl.BlockSpec((bq, 2 * DQK), lambda m, hp, qb: (qb, 0)),
            pl.BlockSpec((bq, 2 * DQK), lambda m, hp, qb: (qb, 0)),
        ],
        out_specs=pl.BlockSpec((1, bq, 2 * DH), lambda m, hp, qb: (m, qb, hp)),
        out_shape=jax.ShapeDtypeStruct((NMOD, S, H * DH), _F32),
        compiler_params=pltpu.CompilerParams(
            dimension_semantics=("parallel", "parallel", "parallel")),
    )(p1, kf, p2, cosq, sinq)


def _ln_res(x, r, gamma, beta):
    g, s, d = x.shape

    def body(x_ref, r_ref, g_ref, b_ref, o_ref):
        y = x_ref[0] + r_ref[0]
        mu = jnp.mean(y, -1, keepdims=True)
        var = jnp.mean((y - mu) ** 2, -1, keepdims=True)
        o_ref[0] = (y - mu) / jnp.sqrt(var + 1e-5) * g_ref[0] + b_ref[0]

    return pl.pallas_call(
        body,
        grid=(g,),
        in_specs=[
            pl.BlockSpec((1, s, d), lambda i: (i, 0, 0)),
            pl.BlockSpec((1, s, d), lambda i: (i, 0, 0)),
            pl.BlockSpec((1, 1, d), lambda i: (i, 0, 0)),
            pl.BlockSpec((1, 1, d), lambda i: (i, 0, 0)),
        ],
        out_specs=pl.BlockSpec((1, s, d), lambda i: (i, 0, 0)),
        out_shape=jax.ShapeDtypeStruct((g, s, d), _F32),
        compiler_params=pltpu.CompilerParams(
            dimension_semantics=("parallel",)),
    )(x, r, gamma, beta)


def _router(x, wg):
    g, s, d = x.shape
    e = wg.shape[2]

    def body(x_ref, w_ref, gw_ref, ls_ref):
        logits = lax.dot_general(
            x_ref[0], w_ref[0], (((1,), (0,)), ((), ())),
            preferred_element_type=_F32)
        mx = jnp.max(logits, -1, keepdims=True)
        ex = jnp.exp(logits - mx)
        probs = ex / jnp.sum(ex, -1, keepdims=True)
        lane = lax.broadcasted_iota(jnp.int32, (s, e), 1)
        m1 = jnp.max(probs, -1, keepdims=True)
        i1 = jnp.min(jnp.where(probs == m1, lane, e), -1, keepdims=True)
        oh1 = lane == i1
        p2 = jnp.where(oh1, -1.0, probs)
        m2 = jnp.max(p2, -1, keepdims=True)
        i2 = jnp.min(jnp.where(p2 == m2, lane, e), -1, keepdims=True)
        oh2 = lane == i2
        tot = m1 + m2
        gw = jnp.where(oh1, m1 / tot, 0.0) + jnp.where(oh2, m2 / tot, 0.0)
        gw_ref[0] = gw
        cnt = oh1.astype(_F32) + oh2.astype(_F32)
        frac = jnp.mean(cnt, 0, keepdims=True)
        pm = jnp.mean(probs, 0, keepdims=True)
        lose = SCALE * e * jnp.sum(frac * pm)
        ls_ref[0, 0, :] = jnp.full((128,), lose, _F32)

    return pl.pallas_call(
        body,
        grid=(g,),
        in_specs=[
            pl.BlockSpec((1, s, d), lambda i: (i, 0, 0)),
            pl.BlockSpec((1, d, e), lambda i: (i, 0, 0)),
        ],
        out_specs=[
            pl.BlockSpec((1, s, e), lambda i: (i, 0, 0)),
            pl.BlockSpec((1, 1, 128), lambda i: (i, 0, 0)),
        ],
        out_shape=[
            jax.ShapeDtypeStruct((g, s, e), _F32),
            jax.ShapeDtypeStruct((g, 1, 128), _F32),
        ],
        compiler_params=pltpu.CompilerParams(
            dimension_semantics=("parallel",)),
    )(x, wg)


def _moe_one(x, gwx, mi, w1, b1, w2, b2, sw1, sb1, sw2, sb2):
    ne = E + SH

    def body(x_ref, gw_ref, w1_ref, b1_ref, w2_ref, b2_ref,
             sw1_ref, sb1_ref, sw2_ref, sb2_ref, o_ref):
        e = pl.program_id(0)

        @pl.when(e == 0)
        def _():
            o_ref[...] = jnp.zeros((S, D), _F32)

        def ffn(w1v, b1v, w2v, b2v):
            h = lax.dot_general(
                x_ref[0], w1v, (((1,), (0,)), ((), ())),
                preferred_element_type=_F32) + b1v
            h = jnp.maximum(h, 0.0)
            return lax.dot_general(
                h, w2v, (((1,), (0,)), ((), ())),
                preferred_element_type=_F32) + b2v

        @pl.when(e < E)
        def _():
            o_ref[...] += gw_ref[0, 0] * ffn(
                w1_ref[0], b1_ref[0], w2_ref[0], b2_ref[0])

        @pl.when(e == E)
        def _():
            o_ref[...] += ffn(sw1_ref[0], sb1_ref[0], sw2_ref[0], sb2_ref[0])

    clamp = lambda e: (jnp.minimum(e, E - 1), 0, 0)
    return pl.pallas_call(
        body,
        grid=(ne,),
        in_specs=[
            pl.BlockSpec((1, S, D), lambda e: (mi, 0, 0)),
            pl.BlockSpec((1, 1, S, 1), lambda e: (mi, e, 0, 0)),
            pl.BlockSpec((1, D, HID), clamp),
            pl.BlockSpec((1, 1, HID), clamp),
            pl.BlockSpec((1, HID, D), clamp),
            pl.BlockSpec((1, 1, D), clamp),
            pl.BlockSpec((1, D, HID), lambda e: (0, 0, 0)),
            pl.BlockSpec((1, 1, HID), lambda e: (0, 0, 0)),
            pl.BlockSpec((1, HID, D), lambda e: (0, 0, 0)),
            pl.BlockSpec((1, 1, D), lambda e: (0, 0, 0)),
        ],
        out_specs=pl.BlockSpec((S, D), lambda e: (0, 0)),
        out_shape=jax.ShapeDtypeStruct((S, D), _F32),
        compiler_params=pltpu.CompilerParams(
            dimension_semantics=("arbitrary",),
            vmem_limit_bytes=63 * 1024 * 1024),
    )(x, gwx, w1, b1, w2, b2, sw1, sb1, sw2, sb2)


def _gate_head(gx, fcw, fcb, seqr):
    def body(x_ref, w_ref, b_ref, sq_ref, o_ref):
        q = lax.dot_general(
            x_ref[0], w_ref[...], (((1,), (0,)), ((), ())),
            preferred_element_type=_F32) + b_ref[...]
        sq = sq_ref[...]
        ar = lax.broadcasted_iota(jnp.int32, (1, S), 1)
        post = jnp.max(jnp.where(sq == TGT, ar, -1))
        pos = jnp.where(post >= 0, post, S - 1)
        rows = lax.broadcasted_iota(jnp.int32, (S, 128), 0)
        grow = jnp.sum(jnp.where(rows == pos, q, 0.0), 0, keepdims=True)
        lane = lax.broadcasted_iota(jnp.int32, (1, 128), 1)
        lg = jnp.where(lane < NM, grow, -1e9)
        mx = jnp.max(lg)
        exq = jnp.exp(lg - mx)
        o_ref[...] = exq / jnp.sum(exq)

    return pl.pallas_call(
        body,
        grid=(1,),
        in_specs=[
            pl.BlockSpec((1, S, D), lambda i: (NM, 0, 0)),
            pl.BlockSpec((D, 128), lambda i: (0, 0)),
            pl.BlockSpec((1, 128), lambda i: (0, 0)),
            pl.BlockSpec((1, S), lambda i: (0, 0)),
        ],
        out_specs=pl.BlockSpec((1, 128), lambda i: (0, 0)),
        out_shape=jax.ShapeDtypeStruct((1, 128), _F32),
    )(gx, fcw, fcb, seqr)


def _vocab_head(g8, xs, ws, bias, bn=256):
    vv = ws[0].shape[1]

    def body(g_ref, x_ref, *rest):
        w_refs = rest[:NM]
        b_ref, o_ref = rest[NM], rest[NM + 1]
        acc = jnp.zeros((1, bn), _F32)
        for mm in range(NM):
            acc = acc + g_ref[mm] * b_ref[mm]
        acc = jnp.broadcast_to(acc, (S, bn))
        for mm in range(NM):
            acc = acc + g_ref[mm] * lax.dot_general(
                x_ref[mm], w_refs[mm][...], (((1,), (0,)), ((), ())),
                preferred_element_type=_F32)
        o_ref[...] = acc

    grid_spec = pltpu.PrefetchScalarGridSpec(
        num_scalar_prefetch=1,
        grid=(vv // bn,),
        in_specs=[pl.BlockSpec((NMOD, S, D), lambda n, g8_: (0, 0, 0))]
        + [pl.BlockSpec((D, bn), lambda n, g8_: (0, n)) for _ in range(NM)]
        + [pl.BlockSpec((NM, 1, bn), lambda n, g8_: (0, 0, n))],
        out_specs=pl.BlockSpec((S, bn), lambda n, g8_: (0, n)),
    )
    return pl.pallas_call(
        body,
        grid_spec=grid_spec,
        out_shape=jax.ShapeDtypeStruct((S, vv), _F32),
        compiler_params=pltpu.CompilerParams(
            dimension_semantics=("parallel",),
            vmem_limit_bytes=63 * 1024 * 1024),
    )(g8, xs, *ws, bias)


def kernel(params, x):
    seq = x[0].astype(jnp.int32)
    models = params["models"]
    gate = params["gate"]
    blocks = [m["layers"][0] for m in models] + [gate["layers"][0]]

    x7 = _sc_gather7(
        [m["emb"] for m in models] + [gate["emb"]], seq).reshape(NMOD, S, D)

    def _wa(b):
        wqh = b["wq"].reshape(D, H, DH)
        wqrh = b["wqr"].reshape(D, H, DR)
        qperm = jnp.concatenate([wqh, wqrh], -1).reshape(D, H * DQK)
        return jnp.concatenate(
            [qperm, jnp.pad(b["wdkv"], ((0, 0), (0, 64))),
             jnp.pad(b["wkr"], ((0, 0), (0, 64)))], axis=1)

    wa = jnp.stack([_wa(b) for b in blocks])
    p1 = _bmm(x7, wa, bn=384)
    wkv = jnp.stack([
        jnp.pad(jnp.concatenate([b["wuk"], b["wuv"]], axis=1),
                ((0, 64), (0, 0))) for b in blocks])
    p2 = _bmm(p1, wkv, bn=512, x_cols=(H * DQK, 256))

    inv = 1.0 / (THETA ** (jnp.arange(0, DR, 2, dtype=_F32) / DR))
    ang = jnp.arange(S, dtype=_F32)[:, None] * inv
    cosr = jnp.repeat(jnp.cos(ang), 2, axis=1)
    sinr = jnp.repeat(jnp.sin(ang), 2, axis=1)
    one_q = jnp.concatenate([jnp.ones((S, DH), _F32), cosr], axis=1)
    zero_q = jnp.concatenate([jnp.zeros((S, DH), _F32), sinr], axis=1)
    cosq = jnp.tile(one_q, (1, 2))
    sinq = jnp.tile(zero_q, (1, 2))

    kf = _k_assemble(p1, p2, cosr, sinr)
    ao = _attention(p1, kf, p2, cosq, sinq)
    wo = jnp.stack([b["wo"] for b in blocks])
    attn = _bmm(ao, wo, bn=256)

    n1g = jnp.stack([b["n1g"] for b in blocks])[:, None, :]
    n1b = jnp.stack([b["n1b"] for b in blocks])[:, None, :]
    xm1 = _ln_res(x7, attn, n1g, n1b)

    wg = jnp.stack([b["wg"] for b in blocks])
    gw, loss = _router(xm1, wg)
    gwx = jnp.concatenate(
        [gw.transpose(0, 2, 1), jnp.ones((NMOD, SH, S), _F32)],
        axis=1)[..., None]
    ff = jnp.stack([
        _moe_one(xm1, gwx, mi,
                 b["w1"], b["b1"][:, None, :], b["w2"], b["b2"][:, None, :],
                 b["sw1"], b["sb1"][:, None, :], b["sw2"], b["sb2"][:, None, :])
        for mi, b in enumerate(blocks)])

    n2g = jnp.stack([b["n2g"] for b in blocks])[:, None, :]
    n2b = jnp.stack([b["n2b"] for b in blocks])[:, None, :]
    xm2 = _ln_res(xm1, ff, n2g, n2b)

    fcw = jnp.pad(gate["fcw"], ((0, 0), (0, 128 - NM)))
    fcb = jnp.pad(gate["fcb"], (0, 128 - NM))[None, :]
    gvec = _gate_head(xm2, fcw, fcb, seq[None, :])
    g8 = gvec[0, :8]

    fc1b = jnp.stack([m["fc1b"] for m in models])[:, None, :]
    fc2b = jnp.stack([m["fc2b"] for m in models])[:, None, :]
    ct = _vocab_head(g8, xm2, [m["fc1w"] for m in models], fc1b)
    nt = _vocab_head(g8, xm2, [m["fc2w"] for m in models], fc2b)

    tl = jnp.sum(loss[:NM, 0, 0])
    return ct[None], nt[None], tl


# speedup vs baseline: 1.3379x; 1.3379x over previous
import functools
import math

import jax
import jax.numpy as jnp
from jax import lax
from jax.experimental import pallas as pl
from jax.experimental.pallas import tpu as pltpu
from jax.experimental.pallas import tpu_sc as plsc

D = 768; H = 12; DH = 64; DC = 192; DR = 64; HID = 1024
E = 8; SH = 1; V = 8192; THETA = 10000.0; SCALE = 0.02
NM = 6; TGT = 10; S = 2048
NMOD = NM + 1
DQK = DH + DR
NP = H // 2

_F32 = jnp.float32


def _sc_gather7(tables, seq):
    nw = 32
    ch = S // nw

    @functools.partial(
        pl.kernel,
        out_type=jax.ShapeDtypeStruct((NMOD * S, D), _F32),
        mesh=plsc.VectorSubcoreMesh(core_axis_name="c", subcore_axis_name="s"),
        scratch_types=[
            pltpu.VMEM((ch,), jnp.int32),
            pltpu.VMEM((ch, D), _F32),
            pltpu.SemaphoreType.DMA,
        ],
    )
    def k(t0, t1, t2, t3, t4, t5, t6, seq_hbm, out_hbm, idx_v, rows_v, sem):
        wid = lax.axis_index("s") * 2 + lax.axis_index("c")
        base = wid * ch
        pltpu.sync_copy(seq_hbm.at[pl.ds(base, ch)], idx_v)
        for m, t in enumerate((t0, t1, t2, t3, t4, t5, t6)):
            pltpu.async_copy(t.at[idx_v], rows_v, sem).wait()
            pltpu.sync_copy(rows_v, out_hbm.at[pl.ds(m * S + base, ch)])

    return k(*tables, seq)


def _bmm(x, w, bn, x_cols=None):
    g, m = x.shape[0], x.shape[1]
    kd = w.shape[1]
    n = w.shape[2]
    xoff = 0 if x_cols is None else x_cols[0] // kd

    def body(x_ref, w_ref, o_ref):
        o_ref[0] = lax.dot_general(
            x_ref[0], w_ref[0], (((1,), (0,)), ((), ())),
            preferred_element_type=_F32)

    return pl.pallas_call(
        body,
        grid=(g, n // bn),
        in_specs=[
            pl.BlockSpec((1, m, kd), lambda i, j: (i, 0, xoff)),
            pl.BlockSpec((1, kd, bn), lambda i, j: (i, 0, j)),
        ],
        out_specs=pl.BlockSpec((1, m, bn), lambda i, j: (i, 0, j)),
        out_shape=jax.ShapeDtypeStruct((g, m, n), _F32),
        compiler_params=pltpu.CompilerParams(
            dimension_semantics=("parallel", "parallel")),
    )(x, w)


def _rot_pairs(x):
    r1 = jnp.roll(x, 1, axis=-1)
    r2 = jnp.roll(x, -1, axis=-1)
    lane = lax.broadcasted_iota(jnp.int32, x.shape, len(x.shape) - 1)
    return jnp.where(lane % 2 == 0, -r2, r1)


def _k_assemble(p1, p2, cosr, sinr):
    def body(kr_ref, kn_ref, c_ref, s_ref, o_ref):
        kr = kr_ref[0][:, :DR]
        kr_ro = kr * c_ref[...] + _rot_pairs(kr) * s_ref[...]
        kn = kn_ref[0]
        parts = []
        for h in range(H):
            parts.append(kn[:, h * DH:(h + 1) * DH])
            parts.append(kr_ro)
        o_ref[0] = jnp.concatenate(parts, axis=-1)

    return pl.pallas_call(
        body,
        grid=(NMOD,),
        in_specs=[
            pl.BlockSpec((1, S, 128), lambda i: (i, 0, 14)),
            pl.BlockSpec((1, S, H * DH), lambda i: (i, 0, 0)),
            pl.BlockSpec((S, DR), lambda i: (0, 0)),
            pl.BlockSpec((S, DR), lambda i: (0, 0)),
        ],
        out_specs=pl.BlockSpec((1, S, H * DQK), lambda i: (i, 0, 0)),
        out_shape=jax.ShapeDtypeStruct((NMOD, S, H * DQK), _F32),
        compiler_params=pltpu.CompilerParams(
            dimension_semantics=("parallel",)),
    )(p1, p2, cosr, sinr)


_BQ = 256


def _attention(p1, kf, p2, cosq, sinq):
    bq = _BQ
    scale = 1.0 / math.sqrt(float(DQK))

    ntier = (S // bq) // 2

    def body(q_ref, k_ref, v_ref, c_ref, s_ref, o_ref):
        qb = pl.program_id(2)
        q = q_ref[0]
        q = q * c_ref[...] + _rot_pairs(q) * s_ref[...]

        for it in range(ntier):
            kl = 2 * bq * (it + 1)

            @pl.when(qb // 2 == it)
            def _(kl=kl):
                row = qb * bq + lax.broadcasted_iota(jnp.int32, (bq, kl), 0)
                col = lax.broadcasted_iota(jnp.int32, (bq, kl), 1)
                neg = jnp.where(col <= row, 0.0, -1e9)
                outs = []
                for t in range(2):
                    sc = lax.dot_general(
                        q[:, t * DQK:(t + 1) * DQK],
                        k_ref[0, :kl, t * DQK:(t + 1) * DQK],
                        (((1,), (1,)), ((), ())),
                        preferred_element_type=_F32) * scale + neg
                    mx = jnp.max(sc, -1, keepdims=True)
                    ex = jnp.exp(sc - mx)
                    p = ex / jnp.sum(ex, -1, keepdims=True)
                    outs.append(lax.dot_general(
                        p, v_ref[0, :kl, t * DH:(t + 1) * DH],
                        (((1,), (0,)), ((), ())),
                        preferred_element_type=_F32))
                o_ref[0] = jnp.concatenate(outs, axis=-1)

    return pl.pallas_call(
        body,
        grid=(NMOD, NP, S // bq),
        in_specs=[
            pl.BlockSpec((1, bq, 2 * DQK), lambda m, hp, qb: (m, qb, hp)),
            pl.BlockSpec((1, S, 2 * DQK), lambda m, hp, qb: (m, 0, hp)),
            pl.BlockSpec((1, S, 2 * DH), lambda m, hp, qb: (m, 0, 6 + hp)),
            pl.BlockSpec((bq, 2 * DQK), lambda m, hp, qb: (qb, 0)),
            pl.BlockSpec((bq, 2 * DQK), lambda m, hp, qb: (qb, 0)),
        ],
        out_specs=pl.BlockSpec((1, bq, 2 * DH), lambda m, hp, qb: (m, qb, hp)),
        out_shape=jax.ShapeDtypeStruct((NMOD, S, H * DH), _F32),
        compiler_params=pltpu.CompilerParams(
            dimension_semantics=("parallel", "parallel", "parallel")),
    )(p1, kf, p2, cosq, sinq)


def _ln_res(x, r, gamma, beta):
    g, s, d = x.shape

    def body(x_ref, r_ref, g_ref, b_ref, o_ref):
        y = x_ref[0] + r_ref[0]
        mu = jnp.mean(y, -1, keepdims=True)
        var = jnp.mean((y - mu) ** 2, -1, keepdims=True)
        o_ref[0] = (y - mu) / jnp.sqrt(var + 1e-5) * g_ref[0] + b_ref[0]

    return pl.pallas_call(
        body,
        grid=(g,),
        in_specs=[
            pl.BlockSpec((1, s, d), lambda i: (i, 0, 0)),
            pl.BlockSpec((1, s, d), lambda i: (i, 0, 0)),
            pl.BlockSpec((1, 1, d), lambda i: (i, 0, 0)),
            pl.BlockSpec((1, 1, d), lambda i: (i, 0, 0)),
        ],
        out_specs=pl.BlockSpec((1, s, d), lambda i: (i, 0, 0)),
        out_shape=jax.ShapeDtypeStruct((g, s, d), _F32),
        compiler_params=pltpu.CompilerParams(
            dimension_semantics=("parallel",)),
    )(x, r, gamma, beta)


def _router(x, wg):
    g, s, d = x.shape
    e = wg.shape[2]

    def body(x_ref, w_ref, gw_ref, ls_ref):
        logits = lax.dot_general(
            x_ref[0], w_ref[0], (((1,), (0,)), ((), ())),
            preferred_element_type=_F32)
        mx = jnp.max(logits, -1, keepdims=True)
        ex = jnp.exp(logits - mx)
        probs = ex / jnp.sum(ex, -1, keepdims=True)
        lane = lax.broadcasted_iota(jnp.int32, (s, e), 1)
        m1 = jnp.max(probs, -1, keepdims=True)
        i1 = jnp.min(jnp.where(probs == m1, lane, e), -1, keepdims=True)
        oh1 = lane == i1
        p2 = jnp.where(oh1, -1.0, probs)
        m2 = jnp.max(p2, -1, keepdims=True)
        i2 = jnp.min(jnp.where(p2 == m2, lane, e), -1, keepdims=True)
        oh2 = lane == i2
        tot = m1 + m2
        gw = jnp.where(oh1, m1 / tot, 0.0) + jnp.where(oh2, m2 / tot, 0.0)
        gw_ref[0] = gw
        cnt = oh1.astype(_F32) + oh2.astype(_F32)
        frac = jnp.mean(cnt, 0, keepdims=True)
        pm = jnp.mean(probs, 0, keepdims=True)
        lose = SCALE * e * jnp.sum(frac * pm)
        ls_ref[0, 0, :] = jnp.full((128,), lose, _F32)

    return pl.pallas_call(
        body,
        grid=(g,),
        in_specs=[
            pl.BlockSpec((1, s, d), lambda i: (i, 0, 0)),
            pl.BlockSpec((1, d, e), lambda i: (i, 0, 0)),
        ],
        out_specs=[
            pl.BlockSpec((1, s, e), lambda i: (i, 0, 0)),
            pl.BlockSpec((1, 1, 128), lambda i: (i, 0, 0)),
        ],
        out_shape=[
            jax.ShapeDtypeStruct((g, s, e), _F32),
            jax.ShapeDtypeStruct((g, 1, 128), _F32),
        ],
        compiler_params=pltpu.CompilerParams(
            dimension_semantics=("parallel",)),
    )(x, wg)


def _moe_one(x, gwx, mi, w1, b1, w2, b2, sw1, sb1, sw2, sb2):
    ne = E + SH

    def body(x_ref, gw_ref, w1_ref, b1_ref, w2_ref, b2_ref,
             sw1_ref, sb1_ref, sw2_ref, sb2_ref, o_ref):
        e = pl.program_id(0)

        @pl.when(e == 0)
        def _():
            o_ref[...] = jnp.zeros((S, D), _F32)

        def ffn(w1v, b1v, w2v, b2v):
            h = lax.dot_general(
                x_ref[0], w1v, (((1,), (0,)), ((), ())),
                preferred_element_type=_F32) + b1v
            h = jnp.maximum(h, 0.0)
            return lax.dot_general(
                h, w2v, (((1,), (0,)), ((), ())),
                preferred_element_type=_F32) + b2v

        @pl.when(e < E)
        def _():
            o_ref[...] += gw_ref[0, 0] * ffn(
                w1_ref[0], b1_ref[0], w2_ref[0], b2_ref[0])

        @pl.when(e == E)
        def _():
            o_ref[...] += ffn(sw1_ref[0], sb1_ref[0], sw2_ref[0], sb2_ref[0])

    clamp = lambda e: (jnp.minimum(e, E - 1), 0, 0)
    return pl.pallas_call(
        body,
        grid=(ne,),
        in_specs=[
            pl.BlockSpec((1, S, D), lambda e: (mi, 0, 0)),
            pl.BlockSpec((1, 1, S, 1), lambda e: (mi, e, 0, 0)),
            pl.BlockSpec((1, D, HID), clamp),
            pl.BlockSpec((1, 1, HID), clamp),
            pl.BlockSpec((1, HID, D), clamp),
            pl.BlockSpec((1, 1, D), clamp),
            pl.BlockSpec((1, D, HID), lambda e: (0, 0, 0)),
            pl.BlockSpec((1, 1, HID), lambda e: (0, 0, 0)),
            pl.BlockSpec((1, HID, D), lambda e: (0, 0, 0)),
            pl.BlockSpec((1, 1, D), lambda e: (0, 0, 0)),
        ],
        out_specs=pl.BlockSpec((S, D), lambda e: (0, 0)),
        out_shape=jax.ShapeDtypeStruct((S, D), _F32),
        compiler_params=pltpu.CompilerParams(
            dimension_semantics=("arbitrary",),
            vmem_limit_bytes=63 * 1024 * 1024),
    )(x, gwx, w1, b1, w2, b2, sw1, sb1, sw2, sb2)


def _gate_head(gx, fcw, fcb, seqr):
    def body(x_ref, w_ref, b_ref, sq_ref, o_ref):
        q = lax.dot_general(
            x_ref[0], w_ref[...], (((1,), (0,)), ((), ())),
            preferred_element_type=_F32) + b_ref[...]
        sq = sq_ref[...]
        ar = lax.broadcasted_iota(jnp.int32, (1, S), 1)
        post = jnp.max(jnp.where(sq == TGT, ar, -1))
        pos = jnp.where(post >= 0, post, S - 1)
        rows = lax.broadcasted_iota(jnp.int32, (S, 128), 0)
        grow = jnp.sum(jnp.where(rows == pos, q, 0.0), 0, keepdims=True)
        lane = lax.broadcasted_iota(jnp.int32, (1, 128), 1)
        lg = jnp.where(lane < NM, grow, -1e9)
        mx = jnp.max(lg)
        exq = jnp.exp(lg - mx)
        o_ref[...] = exq / jnp.sum(exq)

    return pl.pallas_call(
        body,
        grid=(1,),
        in_specs=[
            pl.BlockSpec((1, S, D), lambda i: (NM, 0, 0)),
            pl.BlockSpec((D, 128), lambda i: (0, 0)),
            pl.BlockSpec((1, 128), lambda i: (0, 0)),
            pl.BlockSpec((1, S), lambda i: (0, 0)),
        ],
        out_specs=pl.BlockSpec((1, 128), lambda i: (0, 0)),
        out_shape=jax.ShapeDtypeStruct((1, 128), _F32),
    )(gx, fcw, fcb, seqr)


def _vocab_head(g8, xs, ws, bias, bn=256):
    vv = ws[0].shape[1]

    def body(g_ref, x_ref, *rest):
        w_refs = rest[:NM]
        b_ref, o_ref = rest[NM], rest[NM + 1]
        acc = jnp.zeros((1, bn), _F32)
        for mm in range(NM):
            acc = acc + g_ref[mm] * b_ref[mm]
        acc = jnp.broadcast_to(acc, (S, bn))
        for mm in range(NM):
            acc = acc + g_ref[mm] * lax.dot_general(
                x_ref[mm], w_refs[mm][...], (((1,), (0,)), ((), ())),
                preferred_element_type=_F32)
        o_ref[...] = acc

    grid_spec = pltpu.PrefetchScalarGridSpec(
        num_scalar_prefetch=1,
        grid=(vv // bn,),
        in_specs=[pl.BlockSpec((NMOD, S, D), lambda n, g8_: (0, 0, 0))]
        + [pl.BlockSpec((D, bn), lambda n, g8_: (0, n)) for _ in range(NM)]
        + [pl.BlockSpec((NM, 1, bn), lambda n, g8_: (0, 0, n))],
        out_specs=pl.BlockSpec((S, bn), lambda n, g8_: (0, n)),
    )
    return pl.pallas_call(
        body,
        grid_spec=grid_spec,
        out_shape=jax.ShapeDtypeStruct((S, vv), _F32),
        compiler_params=pltpu.CompilerParams(
            dimension_semantics=("parallel",),
            vmem_limit_bytes=63 * 1024 * 1024),
    )(g8, xs, *ws, bias)


def kernel(params, x):
    seq = x[0].astype(jnp.int32)
    models = params["models"]
    gate = params["gate"]
    blocks = [m["layers"][0] for m in models] + [gate["layers"][0]]

    x7 = _sc_gather7(
        [m["emb"] for m in models] + [gate["emb"]], seq).reshape(NMOD, S, D)

    def _wa(b):
        wqh = b["wq"].reshape(D, H, DH)
        wqrh = b["wqr"].reshape(D, H, DR)
        qperm = jnp.concatenate([wqh, wqrh], -1).reshape(D, H * DQK)
        return jnp.concatenate(
            [qperm, jnp.pad(b["wdkv"], ((0, 0), (0, 64))),
             jnp.pad(b["wkr"], ((0, 0), (0, 64)))], axis=1)

    wa = jnp.stack([_wa(b) for b in blocks])
    p1 = _bmm(x7, wa, bn=384)
    wkv = jnp.stack([
        jnp.pad(jnp.concatenate([b["wuk"], b["wuv"]], axis=1),
                ((0, 64), (0, 0))) for b in blocks])
    p2 = _bmm(p1, wkv, bn=512, x_cols=(H * DQK, 256))

    inv = 1.0 / (THETA ** (jnp.arange(0, DR, 2, dtype=_F32) / DR))
    ang = jnp.arange(S, dtype=_F32)[:, None] * inv
    cosr = jnp.repeat(jnp.cos(ang), 2, axis=1)
    sinr = jnp.repeat(jnp.sin(ang), 2, axis=1)
    one_q = jnp.concatenate([jnp.ones((S, DH), _F32), cosr], axis=1)
    zero_q = jnp.concatenate([jnp.zeros((S, DH), _F32), sinr], axis=1)
    cosq = jnp.tile(one_q, (1, 2))
    sinq = jnp.tile(zero_q, (1, 2))

    kf = _k_assemble(p1, p2, cosr, sinr)
    ao = _attention(p1, kf, p2, cosq, sinq)
    wo = jnp.stack([b["wo"] for b in blocks])
    attn = _bmm(ao, wo, bn=256)

    n1g = jnp.stack([b["n1g"] for b in blocks])[:, None, :]
    n1b = jnp.stack([b["n1b"] for b in blocks])[:, None, :]
    xm1 = _ln_res(x7, attn, n1g, n1b)

    wg = jnp.stack([b["wg"] for b in blocks])
    gw, loss = _router(xm1, wg)
    gwx = jnp.concatenate(
        [gw.transpose(0, 2, 1), jnp.ones((NMOD, SH, S), _F32)],
        axis=1)[..., None]
    ff = jnp.stack([
        _moe_one(xm1, gwx, mi,
                 b["w1"], b["b1"][:, None, :], b["w2"], b["b2"][:, None, :],
                 b["sw1"], b["sb1"][:, None, :], b["sw2"], b["sb2"][:, None, :])
        for mi, b in enumerate(blocks)])

    n2g = jnp.stack([b["n2g"] for b in blocks])[:, None, :]
    n2b = jnp.stack([b["n2b"] for b in blocks])[:, None, :]
    xm2 = _ln_res(xm1, ff, n2g, n2b)

    fcw = jnp.pad(gate["fcw"], ((0, 0), (0, 128 - NM)))
    fcb = jnp.pad(gate["fcb"], (0, 128 - NM))[None, :]
    gvec = _gate_head(xm2, fcw, fcb, seq[None, :])
    g8 = gvec[0, :8]

    fc1b = jnp.stack([m["fc1b"] for m in models])[:, None, :]
    fc2b = jnp.stack([m["fc2b"] for m in models])[:, None, :]
    ct = _vocab_head(g8, xm2, [m["fc1w"] for m in models], fc1b)
    nt = _vocab_head(g8, xm2, [m["fc2w"] for m in models], fc2b)

    tl = jnp.sum(loss[:NM, 0, 0])
    return ct[None], nt[None], tl


# speedup vs baseline: 1.3514x; 1.0101x over previous
import functools
import math

import jax
import jax.numpy as jnp
from jax import lax
from jax.experimental import pallas as pl
from jax.experimental.pallas import tpu as pltpu
from jax.experimental.pallas import tpu_sc as plsc

D = 768; H = 12; DH = 64; DC = 192; DR = 64; HID = 1024
E = 8; SH = 1; V = 8192; THETA = 10000.0; SCALE = 0.02
NM = 6; TGT = 10; S = 2048
NMOD = NM + 1
DQK = DH + DR
NP = H // 2

_F32 = jnp.float32


def _sc_gather7(tables, seq):
    nw = 32
    ch = S // nw

    @functools.partial(
        pl.kernel,
        out_type=jax.ShapeDtypeStruct((NMOD * S, D), _F32),
        mesh=plsc.VectorSubcoreMesh(core_axis_name="c", subcore_axis_name="s"),
        scratch_types=[
            pltpu.VMEM((ch,), jnp.int32),
            pltpu.VMEM((ch, D), _F32),
            pltpu.SemaphoreType.DMA,
        ],
    )
    def k(t0, t1, t2, t3, t4, t5, t6, seq_hbm, out_hbm, idx_v, rows_v, sem):
        wid = lax.axis_index("s") * 2 + lax.axis_index("c")
        base = wid * ch
        pltpu.sync_copy(seq_hbm.at[pl.ds(base, ch)], idx_v)
        for m, t in enumerate((t0, t1, t2, t3, t4, t5, t6)):
            pltpu.async_copy(t.at[idx_v], rows_v, sem).wait()
            pltpu.sync_copy(rows_v, out_hbm.at[pl.ds(m * S + base, ch)])

    return k(*tables, seq)


def _bmm(x, w, bn, x_cols=None):
    g, m = x.shape[0], x.shape[1]
    kd = w.shape[1]
    n = w.shape[2]
    xoff = 0 if x_cols is None else x_cols[0] // kd

    def body(x_ref, w_ref, o_ref):
        o_ref[0] = lax.dot_general(
            x_ref[0], w_ref[0], (((1,), (0,)), ((), ())),
            preferred_element_type=_F32)

    return pl.pallas_call(
        body,
        grid=(g, n // bn),
        in_specs=[
            pl.BlockSpec((1, m, kd), lambda i, j: (i, 0, xoff)),
            pl.BlockSpec((1, kd, bn), lambda i, j: (i, 0, j)),
        ],
        out_specs=pl.BlockSpec((1, m, bn), lambda i, j: (i, 0, j)),
        out_shape=jax.ShapeDtypeStruct((g, m, n), _F32),
        compiler_params=pltpu.CompilerParams(
            dimension_semantics=("parallel", "parallel")),
    )(x, w)


def _rot_pairs(x):
    r1 = jnp.roll(x, 1, axis=-1)
    r2 = jnp.roll(x, -1, axis=-1)
    lane = lax.broadcasted_iota(jnp.int32, x.shape, len(x.shape) - 1)
    return jnp.where(lane % 2 == 0, -r2, r1)


def _k_assemble(p1, p2, cosr, sinr):
    def body(kr_ref, kn_ref, c_ref, s_ref, o_ref):
        kr = kr_ref[0][:, :DR]
        kr_ro = kr * c_ref[...] + _rot_pairs(kr) * s_ref[...]
        kn = kn_ref[0]
        parts = []
        for h in range(H):
            parts.append(kn[:, h * DH:(h + 1) * DH])
            parts.append(kr_ro)
        o_ref[0] = jnp.concatenate(parts, axis=-1)

    return pl.pallas_call(
        body,
        grid=(NMOD,),
        in_specs=[
            pl.BlockSpec((1, S, 128), lambda i: (i, 0, 14)),
            pl.BlockSpec((1, S, H * DH), lambda i: (i, 0, 0)),
            pl.BlockSpec((S, DR), lambda i: (0, 0)),
            pl.BlockSpec((S, DR), lambda i: (0, 0)),
        ],
        out_specs=pl.BlockSpec((1, S, H * DQK), lambda i: (i, 0, 0)),
        out_shape=jax.ShapeDtypeStruct((NMOD, S, H * DQK), _F32),
        compiler_params=pltpu.CompilerParams(
            dimension_semantics=("parallel",)),
    )(p1, p2, cosr, sinr)


_BQ = 256


def _attention(p1, kf, p2, cosq, sinq):
    bq = _BQ
    scale = 1.0 / math.sqrt(float(DQK))

    ntier = S // bq

    def body(q_ref, k_ref, v_ref, c_ref, s_ref, o_ref):
        qb = pl.program_id(2)
        q = q_ref[0]
        q = q * c_ref[...] + _rot_pairs(q) * s_ref[...]

        for it in range(ntier):
            kl = bq * (it + 1)

            @pl.when(qb == it)
            def _(kl=kl):
                row = qb * bq + lax.broadcasted_iota(jnp.int32, (bq, kl), 0)
                col = lax.broadcasted_iota(jnp.int32, (bq, kl), 1)
                neg = jnp.where(col <= row, 0.0, -1e9)
                outs = []
                for t in range(2):
                    sc = lax.dot_general(
                        q[:, t * DQK:(t + 1) * DQK],
                        k_ref[0, :kl, t * DQK:(t + 1) * DQK],
                        (((1,), (1,)), ((), ())),
                        preferred_element_type=_F32) * scale + neg
                    mx = jnp.max(sc, -1, keepdims=True)
                    ex = jnp.exp(sc - mx)
                    p = ex / jnp.sum(ex, -1, keepdims=True)
                    outs.append(lax.dot_general(
                        p, v_ref[0, :kl, t * DH:(t + 1) * DH],
                        (((1,), (0,)), ((), ())),
                        preferred_element_type=_F32))
                o_ref[0] = jnp.concatenate(outs, axis=-1)

    return pl.pallas_call(
        body,
        grid=(NMOD, NP, S // bq),
        in_specs=[
            pl.BlockSpec((1, bq, 2 * DQK), lambda m, hp, qb: (m, qb, hp)),
            pl.BlockSpec((1, S, 2 * DQK), lambda m, hp, qb: (m, 0, hp)),
            pl.BlockSpec((1, S, 2 * DH), lambda m, hp, qb: (m, 0, 6 + hp)),
            pl.BlockSpec((bq, 2 * DQK), lambda m, hp, qb: (qb, 0)),
            pl.BlockSpec((bq, 2 * DQK), lambda m, hp, qb: (qb, 0)),
        ],
        out_specs=pl.BlockSpec((1, bq, 2 * DH), lambda m, hp, qb: (m, qb, hp)),
        out_shape=jax.ShapeDtypeStruct((NMOD, S, H * DH), _F32),
        compiler_params=pltpu.CompilerParams(
            dimension_semantics=("parallel", "parallel", "parallel")),
    )(p1, kf, p2, cosq, sinq)


def _ln_res(x, r, gamma, beta):
    g, s, d = x.shape

    def body(x_ref, r_ref, g_ref, b_ref, o_ref):
        y = x_ref[0] + r_ref[0]
        mu = jnp.mean(y, -1, keepdims=True)
        var = jnp.mean((y - mu) ** 2, -1, keepdims=True)
        o_ref[0] = (y - mu) / jnp.sqrt(var + 1e-5) * g_ref[0] + b_ref[0]

    return pl.pallas_call(
        body,
        grid=(g,),
        in_specs=[
            pl.BlockSpec((1, s, d), lambda i: (i, 0, 0)),
            pl.BlockSpec((1, s, d), lambda i: (i, 0, 0)),
            pl.BlockSpec((1, 1, d), lambda i: (i, 0, 0)),
            pl.BlockSpec((1, 1, d), lambda i: (i, 0, 0)),
        ],
        out_specs=pl.BlockSpec((1, s, d), lambda i: (i, 0, 0)),
        out_shape=jax.ShapeDtypeStruct((g, s, d), _F32),
        compiler_params=pltpu.CompilerParams(
            dimension_semantics=("parallel",)),
    )(x, r, gamma, beta)


def _router(x, wg):
    g, s, d = x.shape
    e = wg.shape[2]

    def body(x_ref, w_ref, gw_ref, ls_ref):
        logits = lax.dot_general(
            x_ref[0], w_ref[0], (((1,), (0,)), ((), ())),
            preferred_element_type=_F32)
        mx = jnp.max(logits, -1, keepdims=True)
        ex = jnp.exp(logits - mx)
        probs = ex / jnp.sum(ex, -1, keepdims=True)
        lane = lax.broadcasted_iota(jnp.int32, (s, e), 1)
        m1 = jnp.max(probs, -1, keepdims=True)
        i1 = jnp.min(jnp.where(probs == m1, lane, e), -1, keepdims=True)
        oh1 = lane == i1
        p2 = jnp.where(oh1, -1.0, probs)
        m2 = jnp.max(p2, -1, keepdims=True)
        i2 = jnp.min(jnp.where(p2 == m2, lane, e), -1, keepdims=True)
        oh2 = lane == i2
        tot = m1 + m2
        gw = jnp.where(oh1, m1 / tot, 0.0) + jnp.where(oh2, m2 / tot, 0.0)
        gw_ref[0] = gw
        cnt = oh1.astype(_F32) + oh2.astype(_F32)
        frac = jnp.mean(cnt, 0, keepdims=True)
        pm = jnp.mean(probs, 0, keepdims=True)
        lose = SCALE * e * jnp.sum(frac * pm)
        ls_ref[0, 0, :] = jnp.full((128,), lose, _F32)

    return pl.pallas_call(
        body,
        grid=(g,),
        in_specs=[
            pl.BlockSpec((1, s, d), lambda i: (i, 0, 0)),
            pl.BlockSpec((1, d, e), lambda i: (i, 0, 0)),
        ],
        out_specs=[
            pl.BlockSpec((1, s, e), lambda i: (i, 0, 0)),
            pl.BlockSpec((1, 1, 128), lambda i: (i, 0, 0)),
        ],
        out_shape=[
            jax.ShapeDtypeStruct((g, s, e), _F32),
            jax.ShapeDtypeStruct((g, 1, 128), _F32),
        ],
        compiler_params=pltpu.CompilerParams(
            dimension_semantics=("parallel",)),
    )(x, wg)


def _moe_one(x, gwx, mi, w1, b1, w2, b2, sw1, sb1, sw2, sb2):
    ne = E + SH

    def body(x_ref, gw_ref, w1_ref, b1_ref, w2_ref, b2_ref,
             sw1_ref, sb1_ref, sw2_ref, sb2_ref, o_ref):
        e = pl.program_id(0)

        @pl.when(e == 0)
        def _():
            o_ref[...] = jnp.zeros((S, D), _F32)

        def ffn(w1v, b1v, w2v, b2v):
            h = lax.dot_general(
                x_ref[0], w1v, (((1,), (0,)), ((), ())),
                preferred_element_type=_F32) + b1v
            h = jnp.maximum(h, 0.0)
            return lax.dot_general(
                h, w2v, (((1,), (0,)), ((), ())),
                preferred_element_type=_F32) + b2v

        @pl.when(e < E)
        def _():
            o_ref[...] += gw_ref[0, 0] * ffn(
                w1_ref[0], b1_ref[0], w2_ref[0], b2_ref[0])

        @pl.when(e == E)
        def _():
            o_ref[...] += ffn(sw1_ref[0], sb1_ref[0], sw2_ref[0], sb2_ref[0])

    clamp = lambda e: (jnp.minimum(e, E - 1), 0, 0)
    return pl.pallas_call(
        body,
        grid=(ne,),
        in_specs=[
            pl.BlockSpec((1, S, D), lambda e: (mi, 0, 0)),
            pl.BlockSpec((1, 1, S, 1), lambda e: (mi, e, 0, 0)),
            pl.BlockSpec((1, D, HID), clamp),
            pl.BlockSpec((1, 1, HID), clamp),
            pl.BlockSpec((1, HID, D), clamp),
            pl.BlockSpec((1, 1, D), clamp),
            pl.BlockSpec((1, D, HID), lambda e: (0, 0, 0)),
            pl.BlockSpec((1, 1, HID), lambda e: (0, 0, 0)),
            pl.BlockSpec((1, HID, D), lambda e: (0, 0, 0)),
            pl.BlockSpec((1, 1, D), lambda e: (0, 0, 0)),
        ],
        out_specs=pl.BlockSpec((S, D), lambda e: (0, 0)),
        out_shape=jax.ShapeDtypeStruct((S, D), _F32),
        compiler_params=pltpu.CompilerParams(
            dimension_semantics=("arbitrary",),
            vmem_limit_bytes=63 * 1024 * 1024),
    )(x, gwx, w1, b1, w2, b2, sw1, sb1, sw2, sb2)


def _gate_head(gx, fcw, fcb, seqr):
    def body(x_ref, w_ref, b_ref, sq_ref, o_ref):
        q = lax.dot_general(
            x_ref[0], w_ref[...], (((1,), (0,)), ((), ())),
            preferred_element_type=_F32) + b_ref[...]
        sq = sq_ref[...]
        ar = lax.broadcasted_iota(jnp.int32, (1, S), 1)
        post = jnp.max(jnp.where(sq == TGT, ar, -1))
        pos = jnp.where(post >= 0, post, S - 1)
        rows = lax.broadcasted_iota(jnp.int32, (S, 128), 0)
        grow = jnp.sum(jnp.where(rows == pos, q, 0.0), 0, keepdims=True)
        lane = lax.broadcasted_iota(jnp.int32, (1, 128), 1)
        lg = jnp.where(lane < NM, grow, -1e9)
        mx = jnp.max(lg)
        exq = jnp.exp(lg - mx)
        o_ref[...] = exq / jnp.sum(exq)

    return pl.pallas_call(
        body,
        grid=(1,),
        in_specs=[
            pl.BlockSpec((1, S, D), lambda i: (NM, 0, 0)),
            pl.BlockSpec((D, 128), lambda i: (0, 0)),
            pl.BlockSpec((1, 128), lambda i: (0, 0)),
            pl.BlockSpec((1, S), lambda i: (0, 0)),
        ],
        out_specs=pl.BlockSpec((1, 128), lambda i: (0, 0)),
        out_shape=jax.ShapeDtypeStruct((1, 128), _F32),
    )(gx, fcw, fcb, seqr)


def _vocab_head(g8, xs, ws, bias, bn=256):
    vv = ws[0].shape[1]

    def body(g_ref, x_ref, *rest):
        w_refs = rest[:NM]
        b_ref, o_ref = rest[NM], rest[NM + 1]
        acc = jnp.zeros((1, bn), _F32)
        for mm in range(NM):
            acc = acc + g_ref[mm] * b_ref[mm]
        acc = jnp.broadcast_to(acc, (S, bn))
        for mm in range(NM):
            acc = acc + g_ref[mm] * lax.dot_general(
                x_ref[mm], w_refs[mm][...], (((1,), (0,)), ((), ())),
                preferred_element_type=_F32)
        o_ref[...] = acc

    grid_spec = pltpu.PrefetchScalarGridSpec(
        num_scalar_prefetch=1,
        grid=(vv // bn,),
        in_specs=[pl.BlockSpec((NMOD, S, D), lambda n, g8_: (0, 0, 0))]
        + [pl.BlockSpec((D, bn), lambda n, g8_: (0, n)) for _ in range(NM)]
        + [pl.BlockSpec((NM, 1, bn), lambda n, g8_: (0, 0, n))],
        out_specs=pl.BlockSpec((S, bn), lambda n, g8_: (0, n)),
    )
    return pl.pallas_call(
        body,
        grid_spec=grid_spec,
        out_shape=jax.ShapeDtypeStruct((S, vv), _F32),
        compiler_params=pltpu.CompilerParams(
            dimension_semantics=("parallel",),
            vmem_limit_bytes=63 * 1024 * 1024),
    )(g8, xs, *ws, bias)


def kernel(params, x):
    seq = x[0].astype(jnp.int32)
    models = params["models"]
    gate = params["gate"]
    blocks = [m["layers"][0] for m in models] + [gate["layers"][0]]

    x7 = _sc_gather7(
        [m["emb"] for m in models] + [gate["emb"]], seq).reshape(NMOD, S, D)

    def _wa(b):
        wqh = b["wq"].reshape(D, H, DH)
        wqrh = b["wqr"].reshape(D, H, DR)
        qperm = jnp.concatenate([wqh, wqrh], -1).reshape(D, H * DQK)
        return jnp.concatenate(
            [qperm, jnp.pad(b["wdkv"], ((0, 0), (0, 64))),
             jnp.pad(b["wkr"], ((0, 0), (0, 64)))], axis=1)

    wa = jnp.stack([_wa(b) for b in blocks])
    p1 = _bmm(x7, wa, bn=384)
    wkv = jnp.stack([
        jnp.pad(jnp.concatenate([b["wuk"], b["wuv"]], axis=1),
                ((0, 64), (0, 0))) for b in blocks])
    p2 = _bmm(p1, wkv, bn=512, x_cols=(H * DQK, 256))

    inv = 1.0 / (THETA ** (jnp.arange(0, DR, 2, dtype=_F32) / DR))
    ang = jnp.arange(S, dtype=_F32)[:, None] * inv
    cosr = jnp.repeat(jnp.cos(ang), 2, axis=1)
    sinr = jnp.repeat(jnp.sin(ang), 2, axis=1)
    one_q = jnp.concatenate([jnp.ones((S, DH), _F32), cosr], axis=1)
    zero_q = jnp.concatenate([jnp.zeros((S, DH), _F32), sinr], axis=1)
    cosq = jnp.tile(one_q, (1, 2))
    sinq = jnp.tile(zero_q, (1, 2))

    kf = _k_assemble(p1, p2, cosr, sinr)
    ao = _attention(p1, kf, p2, cosq, sinq)
    wo = jnp.stack([b["wo"] for b in blocks])
    attn = _bmm(ao, wo, bn=256)

    n1g = jnp.stack([b["n1g"] for b in blocks])[:, None, :]
    n1b = jnp.stack([b["n1b"] for b in blocks])[:, None, :]
    xm1 = _ln_res(x7, attn, n1g, n1b)

    wg = jnp.stack([b["wg"] for b in blocks])
    gw, loss = _router(xm1, wg)
    gwx = jnp.concatenate(
        [gw.transpose(0, 2, 1), jnp.ones((NMOD, SH, S), _F32)],
        axis=1)[..., None]
    ff = jnp.stack([
        _moe_one(xm1, gwx, mi,
                 b["w1"], b["b1"][:, None, :], b["w2"], b["b2"][:, None, :],
                 b["sw1"], b["sb1"][:, None, :], b["sw2"], b["sb2"][:, None, :])
        for mi, b in enumerate(blocks)])

    n2g = jnp.stack([b["n2g"] for b in blocks])[:, None, :]
    n2b = jnp.stack([b["n2b"] for b in blocks])[:, None, :]
    xm2 = _ln_res(xm1, ff, n2g, n2b)

    fcw = jnp.pad(gate["fcw"], ((0, 0), (0, 128 - NM)))
    fcb = jnp.pad(gate["fcb"], (0, 128 - NM))[None, :]
    gvec = _gate_head(xm2, fcw, fcb, seq[None, :])
    g8 = gvec[0, :8]

    fc1b = jnp.stack([m["fc1b"] for m in models])[:, None, :]
    fc2b = jnp.stack([m["fc2b"] for m in models])[:, None, :]
    ct = _vocab_head(g8, xm2, [m["fc1w"] for m in models], fc1b)
    nt = _vocab_head(g8, xm2, [m["fc2w"] for m in models], fc2b)

    tl = jnp.sum(loss[:NM, 0, 0])
    return ct[None], nt[None], tl


# speedup vs baseline: 1.3879x; 1.0270x over previous
import functools
import math

import jax
import jax.numpy as jnp
from jax import lax
from jax.experimental import pallas as pl
from jax.experimental.pallas import tpu as pltpu
from jax.experimental.pallas import tpu_sc as plsc

D = 768; H = 12; DH = 64; DC = 192; DR = 64; HID = 1024
E = 8; SH = 1; V = 8192; THETA = 10000.0; SCALE = 0.02
NM = 6; TGT = 10; S = 2048
NMOD = NM + 1
DQK = DH + DR
NP = H // 2

_F32 = jnp.float32


def _sc_gather7(tables, seq):
    nw = 32
    ch = S // nw

    @functools.partial(
        pl.kernel,
        out_type=jax.ShapeDtypeStruct((NMOD * S, D), _F32),
        mesh=plsc.VectorSubcoreMesh(core_axis_name="c", subcore_axis_name="s"),
        scratch_types=[
            pltpu.VMEM((ch,), jnp.int32),
            pltpu.VMEM((ch, D), _F32),
            pltpu.SemaphoreType.DMA,
        ],
    )
    def k(t0, t1, t2, t3, t4, t5, t6, seq_hbm, out_hbm, idx_v, rows_v, sem):
        wid = lax.axis_index("s") * 2 + lax.axis_index("c")
        base = wid * ch
        pltpu.sync_copy(seq_hbm.at[pl.ds(base, ch)], idx_v)
        for m, t in enumerate((t0, t1, t2, t3, t4, t5, t6)):
            pltpu.async_copy(t.at[idx_v], rows_v, sem).wait()
            pltpu.sync_copy(rows_v, out_hbm.at[pl.ds(m * S + base, ch)])

    return k(*tables, seq)


def _bmm(x, w, bn, x_cols=None):
    g, m = x.shape[0], x.shape[1]
    kd = w.shape[1]
    n = w.shape[2]
    xoff = 0 if x_cols is None else x_cols[0] // kd

    def body(x_ref, w_ref, o_ref):
        o_ref[0] = lax.dot_general(
            x_ref[0], w_ref[0], (((1,), (0,)), ((), ())),
            preferred_element_type=_F32)

    return pl.pallas_call(
        body,
        grid=(g, n // bn),
        in_specs=[
            pl.BlockSpec((1, m, kd), lambda i, j: (i, 0, xoff)),
            pl.BlockSpec((1, kd, bn), lambda i, j: (i, 0, j)),
        ],
        out_specs=pl.BlockSpec((1, m, bn), lambda i, j: (i, 0, j)),
        out_shape=jax.ShapeDtypeStruct((g, m, n), _F32),
        compiler_params=pltpu.CompilerParams(
            dimension_semantics=("parallel", "parallel")),
    )(x, w)


def _rot_pairs(x):
    r1 = jnp.roll(x, 1, axis=-1)
    r2 = jnp.roll(x, -1, axis=-1)
    lane = lax.broadcasted_iota(jnp.int32, x.shape, len(x.shape) - 1)
    return jnp.where(lane % 2 == 0, -r2, r1)


def _k_assemble(p1, p2, cosr, sinr):
    def body(kr_ref, kn_ref, c_ref, s_ref, o_ref):
        kr = kr_ref[0][:, :DR]
        kr_ro = kr * c_ref[...] + _rot_pairs(kr) * s_ref[...]
        kn = kn_ref[0]
        parts = []
        for h in range(H):
            parts.append(kn[:, h * DH:(h + 1) * DH])
            parts.append(kr_ro)
        o_ref[0] = jnp.concatenate(parts, axis=-1)

    return pl.pallas_call(
        body,
        grid=(NMOD,),
        in_specs=[
            pl.BlockSpec((1, S, 128), lambda i: (i, 0, 14)),
            pl.BlockSpec((1, S, H * DH), lambda i: (i, 0, 0)),
            pl.BlockSpec((S, DR), lambda i: (0, 0)),
            pl.BlockSpec((S, DR), lambda i: (0, 0)),
        ],
        out_specs=pl.BlockSpec((1, S, H * DQK), lambda i: (i, 0, 0)),
        out_shape=jax.ShapeDtypeStruct((NMOD, S, H * DQK), _F32),
        compiler_params=pltpu.CompilerParams(
            dimension_semantics=("parallel",)),
    )(p1, p2, cosr, sinr)


_BQ = 512


def _attention(p1, kf, p2, cosq, sinq):
    bq = _BQ
    scale = 1.0 / math.sqrt(float(DQK))

    ntier = S // bq

    def body(q_ref, k_ref, v_ref, c_ref, s_ref, o_ref):
        qb = pl.program_id(2)
        q = q_ref[0]
        q = q * c_ref[...] + _rot_pairs(q) * s_ref[...]

        for it in range(ntier):
            kl = bq * (it + 1)

            @pl.when(qb == it)
            def _(kl=kl):
                row = qb * bq + lax.broadcasted_iota(jnp.int32, (bq, kl), 0)
                col = lax.broadcasted_iota(jnp.int32, (bq, kl), 1)
                neg = jnp.where(col <= row, 0.0, -1e9)
                outs = []
                for t in range(2):
                    sc = lax.dot_general(
                        q[:, t * DQK:(t + 1) * DQK],
                        k_ref[0, :kl, t * DQK:(t + 1) * DQK],
                        (((1,), (1,)), ((), ())),
                        preferred_element_type=_F32) * scale + neg
                    mx = jnp.max(sc, -1, keepdims=True)
                    ex = jnp.exp(sc - mx)
                    p = ex / jnp.sum(ex, -1, keepdims=True)
                    outs.append(lax.dot_general(
                        p, v_ref[0, :kl, t * DH:(t + 1) * DH],
                        (((1,), (0,)), ((), ())),
                        preferred_element_type=_F32))
                o_ref[0] = jnp.concatenate(outs, axis=-1)

    return pl.pallas_call(
        body,
        grid=(NMOD, NP, S // bq),
        in_specs=[
            pl.BlockSpec((1, bq, 2 * DQK), lambda m, hp, qb: (m, qb, hp)),
            pl.BlockSpec((1, S, 2 * DQK), lambda m, hp, qb: (m, 0, hp)),
            pl.BlockSpec((1, S, 2 * DH), lambda m, hp, qb: (m, 0, 6 + hp)),
            pl.BlockSpec((bq, 2 * DQK), lambda m, hp, qb: (qb, 0)),
            pl.BlockSpec((bq, 2 * DQK), lambda m, hp, qb: (qb, 0)),
        ],
        out_specs=pl.BlockSpec((1, bq, 2 * DH), lambda m, hp, qb: (m, qb, hp)),
        out_shape=jax.ShapeDtypeStruct((NMOD, S, H * DH), _F32),
        compiler_params=pltpu.CompilerParams(
            dimension_semantics=("parallel", "parallel", "parallel")),
    )(p1, kf, p2, cosq, sinq)


def _ln_res(x, r, gamma, beta):
    g, s, d = x.shape

    def body(x_ref, r_ref, g_ref, b_ref, o_ref):
        y = x_ref[0] + r_ref[0]
        mu = jnp.mean(y, -1, keepdims=True)
        var = jnp.mean((y - mu) ** 2, -1, keepdims=True)
        o_ref[0] = (y - mu) / jnp.sqrt(var + 1e-5) * g_ref[0] + b_ref[0]

    return pl.pallas_call(
        body,
        grid=(g,),
        in_specs=[
            pl.BlockSpec((1, s, d), lambda i: (i, 0, 0)),
            pl.BlockSpec((1, s, d), lambda i: (i, 0, 0)),
            pl.BlockSpec((1, 1, d), lambda i: (i, 0, 0)),
            pl.BlockSpec((1, 1, d), lambda i: (i, 0, 0)),
        ],
        out_specs=pl.BlockSpec((1, s, d), lambda i: (i, 0, 0)),
        out_shape=jax.ShapeDtypeStruct((g, s, d), _F32),
        compiler_params=pltpu.CompilerParams(
            dimension_semantics=("parallel",)),
    )(x, r, gamma, beta)


def _router(x, wg):
    g, s, d = x.shape
    e = wg.shape[2]

    def body(x_ref, w_ref, gw_ref, ls_ref):
        logits = lax.dot_general(
            x_ref[0], w_ref[0], (((1,), (0,)), ((), ())),
            preferred_element_type=_F32)
        mx = jnp.max(logits, -1, keepdims=True)
        ex = jnp.exp(logits - mx)
        probs = ex / jnp.sum(ex, -1, keepdims=True)
        lane = lax.broadcasted_iota(jnp.int32, (s, e), 1)
        m1 = jnp.max(probs, -1, keepdims=True)
        i1 = jnp.min(jnp.where(probs == m1, lane, e), -1, keepdims=True)
        oh1 = lane == i1
        p2 = jnp.where(oh1, -1.0, probs)
        m2 = jnp.max(p2, -1, keepdims=True)
        i2 = jnp.min(jnp.where(p2 == m2, lane, e), -1, keepdims=True)
        oh2 = lane == i2
        tot = m1 + m2
        gw = jnp.where(oh1, m1 / tot, 0.0) + jnp.where(oh2, m2 / tot, 0.0)
        gw_ref[0] = gw
        cnt = oh1.astype(_F32) + oh2.astype(_F32)
        frac = jnp.mean(cnt, 0, keepdims=True)
        pm = jnp.mean(probs, 0, keepdims=True)
        lose = SCALE * e * jnp.sum(frac * pm)
        ls_ref[0, 0, :] = jnp.full((128,), lose, _F32)

    return pl.pallas_call(
        body,
        grid=(g,),
        in_specs=[
            pl.BlockSpec((1, s, d), lambda i: (i, 0, 0)),
            pl.BlockSpec((1, d, e), lambda i: (i, 0, 0)),
        ],
        out_specs=[
            pl.BlockSpec((1, s, e), lambda i: (i, 0, 0)),
            pl.BlockSpec((1, 1, 128), lambda i: (i, 0, 0)),
        ],
        out_shape=[
            jax.ShapeDtypeStruct((g, s, e), _F32),
            jax.ShapeDtypeStruct((g, 1, 128), _F32),
        ],
        compiler_params=pltpu.CompilerParams(
            dimension_semantics=("parallel",)),
    )(x, wg)


def _moe_one(x, gwx, mi, w1, b1, w2, b2, sw1, sb1, sw2, sb2):
    ne = E + SH

    def body(x_ref, gw_ref, w1_ref, b1_ref, w2_ref, b2_ref,
             sw1_ref, sb1_ref, sw2_ref, sb2_ref, o_ref):
        e = pl.program_id(0)

        @pl.when(e == 0)
        def _():
            o_ref[...] = jnp.zeros((S, D), _F32)

        def ffn(w1v, b1v, w2v, b2v):
            h = lax.dot_general(
                x_ref[0], w1v, (((1,), (0,)), ((), ())),
                preferred_element_type=_F32) + b1v
            h = jnp.maximum(h, 0.0)
            return lax.dot_general(
                h, w2v, (((1,), (0,)), ((), ())),
                preferred_element_type=_F32) + b2v

        @pl.when(e < E)
        def _():
            o_ref[...] += gw_ref[0, 0] * ffn(
                w1_ref[0], b1_ref[0], w2_ref[0], b2_ref[0])

        @pl.when(e == E)
        def _():
            o_ref[...] += ffn(sw1_ref[0], sb1_ref[0], sw2_ref[0], sb2_ref[0])

    clamp = lambda e: (jnp.minimum(e, E - 1), 0, 0)
    return pl.pallas_call(
        body,
        grid=(ne,),
        in_specs=[
            pl.BlockSpec((1, S, D), lambda e: (mi, 0, 0)),
            pl.BlockSpec((1, 1, S, 1), lambda e: (mi, e, 0, 0)),
            pl.BlockSpec((1, D, HID), clamp),
            pl.BlockSpec((1, 1, HID), clamp),
            pl.BlockSpec((1, HID, D), clamp),
            pl.BlockSpec((1, 1, D), clamp),
            pl.BlockSpec((1, D, HID), lambda e: (0, 0, 0)),
            pl.BlockSpec((1, 1, HID), lambda e: (0, 0, 0)),
            pl.BlockSpec((1, HID, D), lambda e: (0, 0, 0)),
            pl.BlockSpec((1, 1, D), lambda e: (0, 0, 0)),
        ],
        out_specs=pl.BlockSpec((S, D), lambda e: (0, 0)),
        out_shape=jax.ShapeDtypeStruct((S, D), _F32),
        compiler_params=pltpu.CompilerParams(
            dimension_semantics=("arbitrary",),
            vmem_limit_bytes=63 * 1024 * 1024),
    )(x, gwx, w1, b1, w2, b2, sw1, sb1, sw2, sb2)


def _gate_head(gx, fcw, fcb, seqr):
    def body(x_ref, w_ref, b_ref, sq_ref, o_ref):
        q = lax.dot_general(
            x_ref[0], w_ref[...], (((1,), (0,)), ((), ())),
            preferred_element_type=_F32) + b_ref[...]
        sq = sq_ref[...]
        ar = lax.broadcasted_iota(jnp.int32, (1, S), 1)
        post = jnp.max(jnp.where(sq == TGT, ar, -1))
        pos = jnp.where(post >= 0, post, S - 1)
        rows = lax.broadcasted_iota(jnp.int32, (S, 128), 0)
        grow = jnp.sum(jnp.where(rows == pos, q, 0.0), 0, keepdims=True)
        lane = lax.broadcasted_iota(jnp.int32, (1, 128), 1)
        lg = jnp.where(lane < NM, grow, -1e9)
        mx = jnp.max(lg)
        exq = jnp.exp(lg - mx)
        o_ref[...] = exq / jnp.sum(exq)

    return pl.pallas_call(
        body,
        grid=(1,),
        in_specs=[
            pl.BlockSpec((1, S, D), lambda i: (NM, 0, 0)),
            pl.BlockSpec((D, 128), lambda i: (0, 0)),
            pl.BlockSpec((1, 128), lambda i: (0, 0)),
            pl.BlockSpec((1, S), lambda i: (0, 0)),
        ],
        out_specs=pl.BlockSpec((1, 128), lambda i: (0, 0)),
        out_shape=jax.ShapeDtypeStruct((1, 128), _F32),
    )(gx, fcw, fcb, seqr)


def _vocab_head(g8, xs, ws, bias, bn=256):
    vv = ws[0].shape[1]

    def body(g_ref, x_ref, *rest):
        w_refs = rest[:NM]
        b_ref, o_ref = rest[NM], rest[NM + 1]
        acc = jnp.zeros((1, bn), _F32)
        for mm in range(NM):
            acc = acc + g_ref[mm] * b_ref[mm]
        acc = jnp.broadcast_to(acc, (S, bn))
        for mm in range(NM):
            acc = acc + g_ref[mm] * lax.dot_general(
                x_ref[mm], w_refs[mm][...], (((1,), (0,)), ((), ())),
                preferred_element_type=_F32)
        o_ref[...] = acc

    grid_spec = pltpu.PrefetchScalarGridSpec(
        num_scalar_prefetch=1,
        grid=(vv // bn,),
        in_specs=[pl.BlockSpec((NMOD, S, D), lambda n, g8_: (0, 0, 0))]
        + [pl.BlockSpec((D, bn), lambda n, g8_: (0, n)) for _ in range(NM)]
        + [pl.BlockSpec((NM, 1, bn), lambda n, g8_: (0, 0, n))],
        out_specs=pl.BlockSpec((S, bn), lambda n, g8_: (0, n)),
    )
    return pl.pallas_call(
        body,
        grid_spec=grid_spec,
        out_shape=jax.ShapeDtypeStruct((S, vv), _F32),
        compiler_params=pltpu.CompilerParams(
            dimension_semantics=("parallel",),
            vmem_limit_bytes=63 * 1024 * 1024),
    )(g8, xs, *ws, bias)


def kernel(params, x):
    seq = x[0].astype(jnp.int32)
    models = params["models"]
    gate = params["gate"]
    blocks = [m["layers"][0] for m in models] + [gate["layers"][0]]

    x7 = _sc_gather7(
        [m["emb"] for m in models] + [gate["emb"]], seq).reshape(NMOD, S, D)

    def _wa(b):
        wqh = b["wq"].reshape(D, H, DH)
        wqrh = b["wqr"].reshape(D, H, DR)
        qperm = jnp.concatenate([wqh, wqrh], -1).reshape(D, H * DQK)
        return jnp.concatenate(
            [qperm, jnp.pad(b["wdkv"], ((0, 0), (0, 64))),
             jnp.pad(b["wkr"], ((0, 0), (0, 64)))], axis=1)

    wa = jnp.stack([_wa(b) for b in blocks])
    p1 = _bmm(x7, wa, bn=384)
    wkv = jnp.stack([
        jnp.pad(jnp.concatenate([b["wuk"], b["wuv"]], axis=1),
                ((0, 64), (0, 0))) for b in blocks])
    p2 = _bmm(p1, wkv, bn=512, x_cols=(H * DQK, 256))

    inv = 1.0 / (THETA ** (jnp.arange(0, DR, 2, dtype=_F32) / DR))
    ang = jnp.arange(S, dtype=_F32)[:, None] * inv
    cosr = jnp.repeat(jnp.cos(ang), 2, axis=1)
    sinr = jnp.repeat(jnp.sin(ang), 2, axis=1)
    one_q = jnp.concatenate([jnp.ones((S, DH), _F32), cosr], axis=1)
    zero_q = jnp.concatenate([jnp.zeros((S, DH), _F32), sinr], axis=1)
    cosq = jnp.tile(one_q, (1, 2))
    sinq = jnp.tile(zero_q, (1, 2))

    kf = _k_assemble(p1, p2, cosr, sinr)
    ao = _attention(p1, kf, p2, cosq, sinq)
    wo = jnp.stack([b["wo"] for b in blocks])
    attn = _bmm(ao, wo, bn=256)

    n1g = jnp.stack([b["n1g"] for b in blocks])[:, None, :]
    n1b = jnp.stack([b["n1b"] for b in blocks])[:, None, :]
    xm1 = _ln_res(x7, attn, n1g, n1b)

    wg = jnp.stack([b["wg"] for b in blocks])
    gw, loss = _router(xm1, wg)
    gwx = jnp.concatenate(
        [gw.transpose(0, 2, 1), jnp.ones((NMOD, SH, S), _F32)],
        axis=1)[..., None]
    ff = jnp.stack([
        _moe_one(xm1, gwx, mi,
                 b["w1"], b["b1"][:, None, :], b["w2"], b["b2"][:, None, :],
                 b["sw1"], b["sb1"][:, None, :], b["sw2"], b["sb2"][:, None, :])
        for mi, b in enumerate(blocks)])

    n2g = jnp.stack([b["n2g"] for b in blocks])[:, None, :]
    n2b = jnp.stack([b["n2b"] for b in blocks])[:, None, :]
    xm2 = _ln_res(xm1, ff, n2g, n2b)

    fcw = jnp.pad(gate["fcw"], ((0, 0), (0, 128 - NM)))
    fcb = jnp.pad(gate["fcb"], (0, 128 - NM))[None, :]
    gvec = _gate_head(xm2, fcw, fcb, seq[None, :])
    g8 = gvec[0, :8]

    fc1b = jnp.stack([m["fc1b"] for m in models])[:, None, :]
    fc2b = jnp.stack([m["fc2b"] for m in models])[:, None, :]
    ct = _vocab_head(g8, xm2, [m["fc1w"] for m in models], fc1b)
    nt = _vocab_head(g8, xm2, [m["fc2w"] for m in models], fc2b)

    tl = jnp.sum(loss[:NM, 0, 0])
    return ct[None], nt[None], tl
